# baseline (speedup 1.0000x reference)
import functools

import jax
import jax.numpy as jnp
from jax import lax
from jax.experimental import pallas as pl
from jax.experimental.pallas import tpu as pltpu
from jax.experimental.pallas import tpu_sc as plsc

_NC = 2
_NS = 16
_NW = _NC * _NS


def _lookup_call(tables_t, users_t, num_fields, vocab, dim, batch):
    mesh = plsc.VectorSubcoreMesh(
        core_axis_name="c", subcore_axis_name="s",
        num_cores=_NC, num_subcores=_NS)

    @functools.partial(
        pl.kernel,
        mesh=mesh,
        out_type=jax.ShapeDtypeStruct((num_fields * dim, batch), jnp.float32),
        scratch_types=[
            pltpu.VMEM((vocab,), jnp.float32),
            pltpu.VMEM((2, batch // 4), jnp.int32),
            pltpu.VMEM((batch,), jnp.float32),
            pltpu.SemaphoreType.DMA((2,)),
            pltpu.SemaphoreType.DMA((4,)),
        ],
        compiler_params=pltpu.CompilerParams(needs_layout_passes=False),
    )
    def lookup_k(t_hbm, u_hbm, out_hbm, drow_v, idx_v, orow_v, isem, osem):
        wid = lax.axis_index("s") * _NC + lax.axis_index("c")
        quart = batch // 4

        def istage(f, q, buf):
            return pltpu.make_async_copy(
                u_hbm.at[f, pl.ds(q * quart, quart)],
                idx_v.at[buf], isem.at[buf])

        def owrite(f, q):
            return pltpu.make_async_copy(
                orow_v.at[pl.ds(q * quart, quart)],
                out_hbm.at[f * dim + wid, pl.ds(q * quart, quart)],
                osem.at[q])

        istage(0, 0, 0).start()

        def fbody(f, _):
            pltpu.sync_copy(t_hbm.at[f, wid], drow_v)
            for q in range(4):
                buf = q % 2
                istage(f, q, buf).wait()
                if q < 3:
                    istage(f, q + 1, 1 - buf).start()
                else:
                    @pl.when(f + 1 < num_fields)
                    def _():
                        istage(f + 1, 0, 1 - buf).start()

                @pl.when(f > 0)
                def _():
                    owrite(f - 1, q).wait()

                def body(j, _, q=q, buf=buf):
                    for t in range(8):
                        u = idx_v[buf, pl.ds(j * 128 + t * 16, 16)]
                        orow_v[pl.ds(q * quart + j * 128 + t * 16, 16)] = (
                            plsc.load_gather(drow_v, [u]))
                    return 0

                lax.fori_loop(0, quart // 128, body, 0)
                owrite(f, q).start()
            return 0

        lax.fori_loop(0, num_fields, fbody, 0)
        for q in range(4):
            owrite(num_fields - 1, q).wait()

    return lookup_k(tables_t, users_t)


def kernel(users, tables):
    num_fields, vocab, dim = tables.shape
    batch = users.shape[0]

    tables_t = jnp.transpose(tables, (0, 2, 1))
    users_t = jnp.transpose(users.astype(jnp.int32), (1, 0))

    out_t = _lookup_call(tables_t, users_t, num_fields, vocab, dim, batch)
    return jnp.transpose(out_t, (1, 0)).reshape(batch, num_fields * dim)


# speedup vs baseline: 1.2205x; 1.2205x over previous
import functools

import jax
import jax.numpy as jnp
from jax import lax
from jax.experimental import pallas as pl
from jax.experimental.pallas import tpu as pltpu
from jax.experimental.pallas import tpu_sc as plsc

_NC = 2
_NS = 16
_NW = _NC * _NS


def _lookup_call(tables_t, users_t, num_fields, vocab, dim, batch):
    mesh = plsc.VectorSubcoreMesh(
        core_axis_name="c", subcore_axis_name="s",
        num_cores=_NC, num_subcores=_NS)

    @functools.partial(
        pl.kernel,
        mesh=mesh,
        out_type=jax.ShapeDtypeStruct((num_fields * dim, batch), jnp.float32),
        scratch_types=[
            pltpu.VMEM((vocab,), jnp.float32),
            pltpu.VMEM((batch // 2,), jnp.int32),
            pltpu.VMEM((batch,), jnp.float32),
            pltpu.SemaphoreType.DMA((2,)),
        ],
        compiler_params=pltpu.CompilerParams(needs_layout_passes=False),
    )
    def lookup_k(t_hbm, u_hbm, out_hbm, drow_v, idx_v, orow_v, osem):
        wid = lax.axis_index("s") * _NC + lax.axis_index("c")
        half = batch // 2

        def owrite(f, h):
            return pltpu.make_async_copy(
                orow_v.at[pl.ds(h * half, half)],
                out_hbm.at[f * dim + wid, pl.ds(h * half, half)],
                osem.at[h])

        for f in range(num_fields):
            pltpu.sync_copy(t_hbm.at[f, wid], drow_v)
            for h in range(2):
                pltpu.sync_copy(u_hbm.at[f, pl.ds(h * half, half)], idx_v)
                if f > 0:
                    owrite(f - 1, h).wait()

                def body(j, _, h=h):
                    for t in range(8):
                        u = idx_v[pl.ds(j * 128 + t * 16, 16)]
                        orow_v[pl.ds(h * half + j * 128 + t * 16, 16)] = (
                            plsc.load_gather(drow_v, [u]))
                    return 0

                lax.fori_loop(0, half // 128, body, 0)
                owrite(f, h).start()
        for h in range(2):
            owrite(num_fields - 1, h).wait()

    return lookup_k(tables_t, users_t)


def kernel(users, tables):
    num_fields, vocab, dim = tables.shape
    batch = users.shape[0]

    tables_t = jnp.transpose(tables, (0, 2, 1))
    users_t = jnp.transpose(users.astype(jnp.int32), (1, 0))

    out_t = _lookup_call(tables_t, users_t, num_fields, vocab, dim, batch)
    return jnp.transpose(out_t, (1, 0)).reshape(batch, num_fields * dim)
